# baseline (speedup 1.0000x reference)
import functools

import jax
import jax.numpy as jnp
from jax import lax
from jax.experimental import pallas as pl
from jax.experimental.pallas import tpu as pltpu
from jax.experimental.pallas import tpu_sc as plsc

N = 50000
E = 800000
D = 100
H = 32
C = 2
H2 = 16

NC, NS = 2, 16
NW = NC * NS
EPW = E // NW
SUB = 250
SUBW = EPW // SUB
STAGE = 10
NOUT = SUBW // STAGE
NROW = 2 * E // SUB
DSTOFF = E // SUB
RPW = 3128
RLAST = N - RPW

BN = 1000
GRID = N // BN

_mesh = plsc.VectorSubcoreMesh(
    core_axis_name="c", subcore_axis_name="s", num_cores=NC, num_subcores=NS
)

_sc_params = pltpu.CompilerParams(use_tc_tiling_on_sc=False)


def _proj1_body(x_ref, wl_ref, wr_ref, y1_ref, xr_ref):
    xb = x_ref[...]
    y1_ref[...] = jnp.dot(xb, wl_ref[...], preferred_element_type=jnp.float32)
    xr_ref[...] = jnp.dot(xb, wr_ref[...], preferred_element_type=jnp.float32)


def _proj1(x, W1l, W1r):
    return pl.pallas_call(
        _proj1_body,
        grid=(GRID,),
        in_specs=[
            pl.BlockSpec((BN, D), lambda i: (i, 0)),
            pl.BlockSpec((D, H), lambda i: (0, 0)),
            pl.BlockSpec((D, H), lambda i: (0, 0)),
        ],
        out_specs=[
            pl.BlockSpec((BN, H), lambda i: (i, 0)),
            pl.BlockSpec((BN, H), lambda i: (i, 0)),
        ],
        out_shape=[
            jax.ShapeDtypeStruct((N, H), jnp.float32),
            jax.ShapeDtypeStruct((N, H), jnp.float32),
        ],
    )(x, W1l, W1r)


def _sc1_body(y1_hbm, ei_hbm, z2_hbm, z1_hbm, ones_hbm,
              acc_out, deg_out,
              srcs, dsts, rowsA, rowsB, onesb, acc_sh, deg_sh, semA, semB):
    c = lax.axis_index("c")
    s = lax.axis_index("s")
    w = c * NS + s

    rbase = jnp.minimum(s * RPW, RLAST)

    pltpu.sync_copy(z2_hbm.at[pl.ds(rbase, RPW)], acc_sh.at[pl.ds(rbase, RPW)])

    @pl.when(s == 0)
    def _():
        pltpu.sync_copy(z1_hbm, deg_sh)

    pltpu.sync_copy(ones_hbm, onesb)
    plsc.subcore_barrier()

    rbufs = ((rowsA, semA), (rowsB, semB))

    @pl.loop(0, NOUT)
    def _(t):
        r0 = w * SUBW + t * STAGE
        pltpu.sync_copy(ei_hbm.at[pl.ds(r0, STAGE)], srcs)
        pltpu.sync_copy(ei_hbm.at[pl.ds(DSTOFF + r0, STAGE)], dsts)
        for k in range(STAGE):
            rk, sk = rbufs[k % 2]
            pltpu.async_copy(y1_hbm.at[srcs.at[k]], rk, sk).wait()
            pltpu.sync_copy(rk, acc_sh.at[dsts.at[k]], add=True)
            pltpu.sync_copy(onesb, deg_sh.at[dsts.at[k]], add=True)

    plsc.subcore_barrier()
    pltpu.sync_copy(acc_sh.at[pl.ds(rbase, RPW)],
                    acc_out.at[c, pl.ds(rbase, RPW)])

    @pl.when(s == 0)
    def _():
        pltpu.sync_copy(deg_sh, deg_out.at[c])


_sc1 = functools.partial(
    pl.kernel,
    out_type=(
        jax.ShapeDtypeStruct((NC, N, H), jnp.float32),
        jax.ShapeDtypeStruct((NC, N), jnp.float32),
    ),
    mesh=_mesh,
    compiler_params=_sc_params,
    scratch_types=[
        pltpu.VMEM((STAGE, SUB), jnp.int32),
        pltpu.VMEM((STAGE, SUB), jnp.int32),
        pltpu.VMEM((SUB, H), jnp.float32),
        pltpu.VMEM((SUB, H), jnp.float32),
        pltpu.VMEM((SUB,), jnp.float32),
        pltpu.VMEM_SHARED((N, H), jnp.float32),
        pltpu.VMEM_SHARED((N,), jnp.float32),
        pltpu.SemaphoreType.DMA,
        pltpu.SemaphoreType.DMA,
    ],
)(_sc1_body)


def _mid_body(p_ref, dg_ref, xr_ref, b1_ref, w2l_ref, w2r_ref, b2_ref,
              y2_ref, zr_ref, di_ref):
    acc = p_ref[0] + p_ref[1]
    deg = dg_ref[0] + dg_ref[1]
    di = 1.0 / jnp.maximum(deg, 1.0)
    h = jnp.maximum(acc * di + b1_ref[...] + xr_ref[...], 0.0)
    y2_ref[...] = jnp.dot(h, w2l_ref[...], preferred_element_type=jnp.float32)
    zr_ref[...] = (
        jnp.dot(h, w2r_ref[...], preferred_element_type=jnp.float32) + b2_ref[...]
    )
    di_ref[...] = di


def _mid(accp, degp, xr, b1, W2lp, W2rp, b2p):
    return pl.pallas_call(
        _mid_body,
        grid=(GRID,),
        in_specs=[
            pl.BlockSpec((NC, BN, H), lambda i: (0, i, 0)),
            pl.BlockSpec((NC, BN, 1), lambda i: (0, i, 0)),
            pl.BlockSpec((BN, H), lambda i: (i, 0)),
            pl.BlockSpec((1, H), lambda i: (0, 0)),
            pl.BlockSpec((H, H2), lambda i: (0, 0)),
            pl.BlockSpec((H, H2), lambda i: (0, 0)),
            pl.BlockSpec((1, H2), lambda i: (0, 0)),
        ],
        out_specs=[
            pl.BlockSpec((BN, H2), lambda i: (i, 0)),
            pl.BlockSpec((BN, H2), lambda i: (i, 0)),
            pl.BlockSpec((BN, 1), lambda i: (i, 0)),
        ],
        out_shape=[
            jax.ShapeDtypeStruct((N, H2), jnp.float32),
            jax.ShapeDtypeStruct((N, H2), jnp.float32),
            jax.ShapeDtypeStruct((N, 1), jnp.float32),
        ],
    )(accp, degp, xr, b1, W2lp, W2rp, b2p)


def _sc2_body(y2_hbm, ei_hbm, z2_hbm,
              acc_out,
              srcs, dsts, rowsA, rowsB, acc_sh, semA, semB):
    c = lax.axis_index("c")
    s = lax.axis_index("s")
    w = c * NS + s

    rbase = jnp.minimum(s * RPW, RLAST)
    pltpu.sync_copy(z2_hbm.at[pl.ds(rbase, RPW)], acc_sh.at[pl.ds(rbase, RPW)])
    plsc.subcore_barrier()

    rbufs = ((rowsA, semA), (rowsB, semB))

    @pl.loop(0, NOUT)
    def _(t):
        r0 = w * SUBW + t * STAGE
        pltpu.sync_copy(ei_hbm.at[pl.ds(r0, STAGE)], srcs)
        pltpu.sync_copy(ei_hbm.at[pl.ds(DSTOFF + r0, STAGE)], dsts)
        for k in range(STAGE):
            rk, sk = rbufs[k % 2]
            pltpu.async_copy(y2_hbm.at[srcs.at[k]], rk, sk).wait()
            pltpu.sync_copy(rk, acc_sh.at[dsts.at[k]], add=True)

    plsc.subcore_barrier()
    pltpu.sync_copy(acc_sh.at[pl.ds(rbase, RPW)],
                    acc_out.at[c, pl.ds(rbase, RPW)])


_sc2 = functools.partial(
    pl.kernel,
    out_type=jax.ShapeDtypeStruct((NC, N, H2), jnp.float32),
    mesh=_mesh,
    compiler_params=_sc_params,
    scratch_types=[
        pltpu.VMEM((STAGE, SUB), jnp.int32),
        pltpu.VMEM((STAGE, SUB), jnp.int32),
        pltpu.VMEM((SUB, H2), jnp.float32),
        pltpu.VMEM((SUB, H2), jnp.float32),
        pltpu.VMEM_SHARED((N, H2), jnp.float32),
        pltpu.SemaphoreType.DMA,
        pltpu.SemaphoreType.DMA,
    ],
)(_sc2_body)


def _out_body(p2_ref, di_ref, zr_ref, o_ref):
    a2 = (p2_ref[0] + p2_ref[1]) * di_ref[...]
    logits = a2 + zr_ref[...]
    l2 = logits[:, 0:C]
    m = jnp.max(l2, axis=1, keepdims=True)
    lse = m + jnp.log(jnp.sum(jnp.exp(l2 - m), axis=1, keepdims=True))
    o_ref[...] = l2 - lse


def _outk(acc2p, di, zr):
    return pl.pallas_call(
        _out_body,
        grid=(GRID,),
        in_specs=[
            pl.BlockSpec((NC, BN, H2), lambda i: (0, i, 0)),
            pl.BlockSpec((BN, 1), lambda i: (i, 0)),
            pl.BlockSpec((BN, H2), lambda i: (i, 0)),
        ],
        out_specs=pl.BlockSpec((BN, C), lambda i: (i, 0)),
        out_shape=jax.ShapeDtypeStruct((N, C), jnp.float32),
    )(acc2p, di, zr)


def kernel(x, edge_index, W1l, b1, W1r, W2l, b2, W2r):
    if edge_index.dtype != jnp.int32:
        edge_index = edge_index.astype(jnp.int32)
    ei2 = edge_index.reshape(NROW, SUB)

    y1, xr = _proj1(x, W1l, W1r)

    z2 = jnp.zeros((N, H), jnp.float32)
    z1 = jnp.zeros((N,), jnp.float32)
    ones = jnp.ones((SUB,), jnp.float32)
    accp, degp = _sc1(y1, ei2, z2, z1, ones)

    W2lp = jnp.pad(W2l, ((0, 0), (0, H2 - C)))
    W2rp = jnp.pad(W2r, ((0, 0), (0, H2 - C)))
    b2p = jnp.pad(b2, (0, H2 - C)).reshape(1, H2)
    y2, zr, di = _mid(accp, degp.reshape(NC, N, 1), xr,
                      b1.reshape(1, H), W2lp, W2rp, b2p)

    z216 = jnp.zeros((N, H2), jnp.float32)
    acc2p = _sc2(y2, ei2, z216)

    return _outk(acc2p, di, zr)


# speedup vs baseline: 1.1136x; 1.1136x over previous
import functools

import jax
import jax.numpy as jnp
from jax import lax
from jax.experimental import pallas as pl
from jax.experimental.pallas import tpu as pltpu
from jax.experimental.pallas import tpu_sc as plsc

N = 50000
E = 800000
D = 100
H = 32
C = 2
H2 = 16

NC, NS = 2, 16
NW = NC * NS
EPW = E // NW
SUB = 500
SUBW = EPW // SUB
STAGE = 10
NOUT = SUBW // STAGE
NROW = 2 * E // SUB
DSTOFF = E // SUB
RPW = 3128
RLAST = N - RPW

BN = 1000
GRID = N // BN

_mesh = plsc.VectorSubcoreMesh(
    core_axis_name="c", subcore_axis_name="s", num_cores=NC, num_subcores=NS
)

_sc_params = pltpu.CompilerParams(use_tc_tiling_on_sc=False)


def _proj1_body(x_ref, wl_ref, wr_ref, y1_ref, xr_ref):
    xb = x_ref[...]
    y1_ref[...] = jnp.dot(xb, wl_ref[...], preferred_element_type=jnp.float32)
    xr_ref[...] = jnp.dot(xb, wr_ref[...], preferred_element_type=jnp.float32)


def _proj1(x, W1l, W1r):
    return pl.pallas_call(
        _proj1_body,
        grid=(GRID,),
        in_specs=[
            pl.BlockSpec((BN, D), lambda i: (i, 0)),
            pl.BlockSpec((D, H), lambda i: (0, 0)),
            pl.BlockSpec((D, H), lambda i: (0, 0)),
        ],
        out_specs=[
            pl.BlockSpec((BN, H), lambda i: (i, 0)),
            pl.BlockSpec((BN, H), lambda i: (i, 0)),
        ],
        out_shape=[
            jax.ShapeDtypeStruct((N, H), jnp.float32),
            jax.ShapeDtypeStruct((N, H), jnp.float32),
        ],
    )(x, W1l, W1r)


def _sc1_body(y1_hbm, ei_hbm, z2_hbm, z1_hbm, ones_hbm,
              acc_out, deg_out,
              srcs, dsts, rows, onesb, acc_sh, deg_sh, sem):
    c = lax.axis_index("c")
    s = lax.axis_index("s")
    w = c * NS + s

    rbase = jnp.minimum(s * RPW, RLAST)

    pltpu.sync_copy(z2_hbm.at[pl.ds(rbase, RPW)], acc_sh.at[pl.ds(rbase, RPW)])

    @pl.when(s == 0)
    def _():
        pltpu.sync_copy(z1_hbm, deg_sh)

    pltpu.sync_copy(ones_hbm, onesb)
    plsc.subcore_barrier()

    @pl.loop(0, NOUT)
    def _(t):
        r0 = w * SUBW + t * STAGE
        pltpu.sync_copy(ei_hbm.at[pl.ds(r0, STAGE)], srcs)
        pltpu.sync_copy(ei_hbm.at[pl.ds(DSTOFF + r0, STAGE)], dsts)
        for k in range(STAGE):
            pltpu.async_copy(y1_hbm.at[srcs.at[k]], rows, sem).wait()
            pltpu.sync_copy(rows, acc_sh.at[dsts.at[k]], add=True)
            pltpu.sync_copy(onesb, deg_sh.at[dsts.at[k]], add=True)

    plsc.subcore_barrier()
    pltpu.sync_copy(acc_sh.at[pl.ds(rbase, RPW)],
                    acc_out.at[c, pl.ds(rbase, RPW)])

    @pl.when(s == 0)
    def _():
        pltpu.sync_copy(deg_sh, deg_out.at[c])


_sc1 = functools.partial(
    pl.kernel,
    out_type=(
        jax.ShapeDtypeStruct((NC, N, H), jnp.float32),
        jax.ShapeDtypeStruct((NC, N), jnp.float32),
    ),
    mesh=_mesh,
    compiler_params=_sc_params,
    scratch_types=[
        pltpu.VMEM((STAGE, SUB), jnp.int32),
        pltpu.VMEM((STAGE, SUB), jnp.int32),
        pltpu.VMEM((SUB, H), jnp.float32),
        pltpu.VMEM((SUB,), jnp.float32),
        pltpu.VMEM_SHARED((N, H), jnp.float32),
        pltpu.VMEM_SHARED((N,), jnp.float32),
        pltpu.SemaphoreType.DMA,
    ],
)(_sc1_body)


def _mid_body(p_ref, dg_ref, xr_ref, b1_ref, w2l_ref, w2r_ref, b2_ref,
              y2_ref, zr_ref, di_ref):
    acc = p_ref[0] + p_ref[1]
    deg = dg_ref[0] + dg_ref[1]
    di = 1.0 / jnp.maximum(deg, 1.0)
    h = jnp.maximum(acc * di + b1_ref[...] + xr_ref[...], 0.0)
    y2_ref[...] = jnp.dot(h, w2l_ref[...], preferred_element_type=jnp.float32)
    zr_ref[...] = (
        jnp.dot(h, w2r_ref[...], preferred_element_type=jnp.float32) + b2_ref[...]
    )
    di_ref[...] = di


def _mid(accp, degp, xr, b1, W2lp, W2rp, b2p):
    return pl.pallas_call(
        _mid_body,
        grid=(GRID,),
        in_specs=[
            pl.BlockSpec((NC, BN, H), lambda i: (0, i, 0)),
            pl.BlockSpec((NC, BN, 1), lambda i: (0, i, 0)),
            pl.BlockSpec((BN, H), lambda i: (i, 0)),
            pl.BlockSpec((1, H), lambda i: (0, 0)),
            pl.BlockSpec((H, H2), lambda i: (0, 0)),
            pl.BlockSpec((H, H2), lambda i: (0, 0)),
            pl.BlockSpec((1, H2), lambda i: (0, 0)),
        ],
        out_specs=[
            pl.BlockSpec((BN, H2), lambda i: (i, 0)),
            pl.BlockSpec((BN, H2), lambda i: (i, 0)),
            pl.BlockSpec((BN, 1), lambda i: (i, 0)),
        ],
        out_shape=[
            jax.ShapeDtypeStruct((N, H2), jnp.float32),
            jax.ShapeDtypeStruct((N, H2), jnp.float32),
            jax.ShapeDtypeStruct((N, 1), jnp.float32),
        ],
    )(accp, degp, xr, b1, W2lp, W2rp, b2p)


def _sc2_body(y2_hbm, ei_hbm, z2_hbm,
              acc_out,
              srcs, dsts, rows, acc_sh, sem):
    c = lax.axis_index("c")
    s = lax.axis_index("s")
    w = c * NS + s

    rbase = jnp.minimum(s * RPW, RLAST)
    pltpu.sync_copy(z2_hbm.at[pl.ds(rbase, RPW)], acc_sh.at[pl.ds(rbase, RPW)])
    plsc.subcore_barrier()

    @pl.loop(0, NOUT)
    def _(t):
        r0 = w * SUBW + t * STAGE
        pltpu.sync_copy(ei_hbm.at[pl.ds(r0, STAGE)], srcs)
        pltpu.sync_copy(ei_hbm.at[pl.ds(DSTOFF + r0, STAGE)], dsts)
        for k in range(STAGE):
            pltpu.async_copy(y2_hbm.at[srcs.at[k]], rows, sem).wait()
            pltpu.sync_copy(rows, acc_sh.at[dsts.at[k]], add=True)

    plsc.subcore_barrier()
    pltpu.sync_copy(acc_sh.at[pl.ds(rbase, RPW)],
                    acc_out.at[c, pl.ds(rbase, RPW)])


_sc2 = functools.partial(
    pl.kernel,
    out_type=jax.ShapeDtypeStruct((NC, N, H2), jnp.float32),
    mesh=_mesh,
    compiler_params=_sc_params,
    scratch_types=[
        pltpu.VMEM((STAGE, SUB), jnp.int32),
        pltpu.VMEM((STAGE, SUB), jnp.int32),
        pltpu.VMEM((SUB, H2), jnp.float32),
        pltpu.VMEM_SHARED((N, H2), jnp.float32),
        pltpu.SemaphoreType.DMA,
    ],
)(_sc2_body)


def _out_body(p2_ref, di_ref, zr_ref, o_ref):
    a2 = (p2_ref[0] + p2_ref[1]) * di_ref[...]
    logits = a2 + zr_ref[...]
    l2 = logits[:, 0:C]
    m = jnp.max(l2, axis=1, keepdims=True)
    lse = m + jnp.log(jnp.sum(jnp.exp(l2 - m), axis=1, keepdims=True))
    o_ref[...] = l2 - lse


def _outk(acc2p, di, zr):
    return pl.pallas_call(
        _out_body,
        grid=(GRID,),
        in_specs=[
            pl.BlockSpec((NC, BN, H2), lambda i: (0, i, 0)),
            pl.BlockSpec((BN, 1), lambda i: (i, 0)),
            pl.BlockSpec((BN, H2), lambda i: (i, 0)),
        ],
        out_specs=pl.BlockSpec((BN, C), lambda i: (i, 0)),
        out_shape=jax.ShapeDtypeStruct((N, C), jnp.float32),
    )(acc2p, di, zr)


def kernel(x, edge_index, W1l, b1, W1r, W2l, b2, W2r):
    if edge_index.dtype != jnp.int32:
        edge_index = edge_index.astype(jnp.int32)
    ei2 = edge_index.reshape(NROW, SUB)

    y1, xr = _proj1(x, W1l, W1r)

    z2 = jnp.zeros((N, H), jnp.float32)
    z1 = jnp.zeros((N,), jnp.float32)
    ones = jnp.ones((SUB,), jnp.float32)
    accp, degp = _sc1(y1, ei2, z2, z1, ones)

    W2lp = jnp.pad(W2l, ((0, 0), (0, H2 - C)))
    W2rp = jnp.pad(W2r, ((0, 0), (0, H2 - C)))
    b2p = jnp.pad(b2, (0, H2 - C)).reshape(1, H2)
    y2, zr, di = _mid(accp, degp.reshape(NC, N, 1), xr,
                      b1.reshape(1, H), W2lp, W2rp, b2p)

    z216 = jnp.zeros((N, H2), jnp.float32)
    acc2p = _sc2(y2, ei2, z216)

    return _outk(acc2p, di, zr)
